# Initial kernel scaffold; baseline (speedup 1.0000x reference)
import jax
import jax.numpy as jnp
from jax.experimental import pallas as pl


def kernel(t, x, edge_index, edge_vals):
    raise NotImplementedError("write your pallas kernel here")


# speedup vs baseline: 4.3363x; 4.3363x over previous
import functools
import jax
import jax.numpy as jnp
from jax import lax
from jax.experimental import pallas as pl
from jax.experimental.pallas import tpu as pltpu
from jax.experimental.pallas import tpu_sc as plsc

NN = 10000
NP = 10240
EE = 320000
DD = 128
NSUB = 16
LANES = 16
CV = DD // LANES
ROWS_PER_TILE = NP // NSUB
RCHUNK = 64
EDGES_PER_TILE = EE // 2 // NSUB
ECHUNK = 80
NCHUNKS = EDGES_PER_TILE // ECHUNK

B_C = 0.1
K_C = 5.0
C_C = 1.0
D_C = 5.0
E_C = 0.9
H_C = 0.1


def _sc_body(xs_hbm, di_hbm, sj_hbm, ev_hbm, out_hbm,
             facc, idxi, idxj, evals, xi, xj, xb, fb,
             sem1, sem2):
    c = lax.axis_index("c")
    s = lax.axis_index("s")
    row0 = s * ROWS_PER_TILE

    def zrow(r, _):
        for g in range(CV):
            xb[r, pl.ds(g * LANES, LANES)] = jnp.zeros((LANES,), jnp.float32)
        return 0
    lax.fori_loop(0, RCHUNK, zrow, 0)
    for k in range(ROWS_PER_TILE // RCHUNK):
        pltpu.sync_copy(xb, facc.at[pl.ds(row0 + k * RCHUNK, RCHUNK)])
    plsc.subcore_barrier()

    e0 = (c * NSUB + s) * EDGES_PER_TILE

    def chunk(k, _):
        eoff = e0 + k * ECHUNK
        pltpu.sync_copy(di_hbm.at[pl.ds(eoff, ECHUNK)], idxi)
        pltpu.sync_copy(sj_hbm.at[pl.ds(eoff, ECHUNK)], idxj)
        pltpu.sync_copy(ev_hbm.at[pl.ds(eoff, ECHUNK)], evals)
        cp1 = pltpu.async_copy(xs_hbm.at[idxi], xi, sem1)
        cp2 = pltpu.async_copy(xs_hbm.at[idxj], xj, sem2)
        cp1.wait()
        cp2.wait()

        def edge_grp(g16, _):
            ebase = g16 * LANES
            av = evals[pl.ds(ebase, LANES)]
            for l in range(LANES):
                a = av[l]
                e2 = ebase + l
                for g in range(CV):
                    sl = pl.ds(g * LANES, LANES)
                    vi = xi[e2, sl]
                    vj = xj[e2, sl]
                    xi[e2, sl] = (a * vi * vj) / (
                        D_C + E_C * vi + H_C * vj)
            return 0
        lax.fori_loop(0, ECHUNK // LANES, edge_grp, 0)
        pltpu.sync_copy(xi, facc.at[idxi], add=True)
        return 0
    lax.fori_loop(0, NCHUNKS, chunk, 0)
    plsc.subcore_barrier()

    m = (c == 0).astype(jnp.float32)
    for k in range(ROWS_PER_TILE // RCHUNK):
        base = row0 + k * RCHUNK
        pltpu.sync_copy(facc.at[pl.ds(base, RCHUNK)], fb)
        pltpu.sync_copy(xs_hbm.at[pl.ds(base, RCHUNK)], xb)

        def wrow(r, _):
            for g in range(CV):
                sl = pl.ds(g * LANES, LANES)
                v = xb[r, sl]
                fb[r, sl] = fb[r, sl] + m * (
                    B_C + v * (1.0 - v * (1.0 / K_C)) * (v * (1.0 / C_C) - 1.0))
            return 0
        lax.fori_loop(0, RCHUNK, wrow, 0)
        pltpu.sync_copy(fb, out_hbm.at[c, pl.ds(base, RCHUNK)])


_mutual_sc = functools.partial(
    pl.kernel,
    out_type=jax.ShapeDtypeStruct((2, NP, DD), jnp.float32),
    mesh=plsc.VectorSubcoreMesh(core_axis_name="c", subcore_axis_name="s"),
    scratch_types=[
        pltpu.VMEM_SHARED((NP, DD), jnp.float32),
        pltpu.VMEM((ECHUNK,), jnp.int32),
        pltpu.VMEM((ECHUNK,), jnp.int32),
        pltpu.VMEM((ECHUNK,), jnp.float32),
        pltpu.VMEM((ECHUNK, DD), jnp.float32),
        pltpu.VMEM((ECHUNK, DD), jnp.float32),
        pltpu.VMEM((RCHUNK, DD), jnp.float32),
        pltpu.VMEM((RCHUNK, DD), jnp.float32),
        pltpu.SemaphoreType.DMA,
        pltpu.SemaphoreType.DMA,
    ],
)(_sc_body)


def kernel(t, x, edge_index, edge_vals):
    pad = jnp.zeros((NP - NN, DD), jnp.float32)
    xs = jnp.concatenate([x, pad], axis=0)
    di = edge_index[0].astype(jnp.int32)
    sj = edge_index[1].astype(jnp.int32)
    out = _mutual_sc(xs, di, sj, edge_vals)
    return out[0, :NN] + out[1, :NN]
